# baseline (speedup 1.0000x reference)
import functools

import jax
import jax.numpy as jnp
from jax import lax
from jax.experimental import pallas as pl
from jax.experimental.pallas import tpu as pltpu
from jax.experimental.pallas import tpu_sc as plsc


@functools.lru_cache(maxsize=None)
def _make_kernel(batch: int, fields: int, vocab: int, dim: int):
    info = plsc.get_sparse_core_info()
    num_cores, num_subcores = info.num_cores, info.num_subcores
    num_workers = num_cores * num_subcores
    lanes = 16
    bt = 128
    n_btiles = batch // bt
    ct_per_w = n_btiles // num_workers
    octets = dim // 8
    groups = bt // lanes
    n_out = batch * fields * dim

    mesh = plsc.VectorSubcoreMesh(core_axis_name="c", subcore_axis_name="s")

    @functools.partial(
        pl.kernel,
        out_type=jax.ShapeDtypeStruct((n_out,), jnp.float32),
        mesh=mesh,
        scratch_types=[
            pltpu.VMEM((fields * ct_per_w * bt,), jnp.int32),
            pltpu.VMEM((ct_per_w, bt, dim), jnp.float32),
            pltpu.VMEM((ct_per_w, octets, 8 * bt), jnp.float32),
            pltpu.SemaphoreType.DMA,
        ]
        + [pltpu.SemaphoreType.DMA for _ in range(2 * ct_per_w)],
        compiler_params=pltpu.CompilerParams(
            use_tc_tiling_on_sc=False, needs_layout_passes=False
        ),
    )
    def emb_kernel(idx_hbm, table_hbm, out_hbm, idx_v, rows_v, ostage, sem_i, *sems):
        gsems, psems = sems[:ct_per_w], sems[ct_per_w:]
        wid = lax.axis_index("s") * num_cores + lax.axis_index("c")
        cb = wid * ct_per_w

        stage = []
        for f in range(fields):
            stage.append(
                pltpu.async_copy(
                    idx_hbm.at[pl.ds(f * batch + cb * bt, ct_per_w * bt)],
                    idx_v.at[pl.ds(f * ct_per_w * bt, ct_per_w * bt)],
                    sem_i,
                )
            )
        for d in stage:
            d.wait()

        iota = lax.iota(jnp.int32, lanes)

        def body(f, carry):
            gd = []
            for cc in range(ct_per_w):
                gd.append(
                    pltpu.async_copy(
                        table_hbm.at[
                            idx_v.at[pl.ds(f * ct_per_w * bt + cc * bt, bt)]
                        ],
                        rows_v.at[cc],
                        gsems[cc],
                    )
                )
            pd = []
            for cc in range(ct_per_w):
                gd[cc].wait()
                for g in range(groups):
                    rowvec = iota + (g * lanes)
                    for d in range(dim):
                        colvec = jnp.full((lanes,), d, jnp.int32)
                        v = plsc.load_gather(rows_v.at[cc], [rowvec, colvec])
                        ostage[cc, d // 8, pl.ds((d % 8) * bt + g * lanes, lanes)] = v
                for r in range(octets):
                    off = (
                        f * (octets * n_btiles * 8 * bt)
                        + r * (n_btiles * 8 * bt)
                        + (cb + cc) * (8 * bt)
                    )
                    pd.append(
                        pltpu.async_copy(
                            ostage.at[cc, r],
                            out_hbm.at[pl.ds(off, 8 * bt)],
                            psems[cc],
                        )
                    )
            for d in pd:
                d.wait()
            return carry

        lax.fori_loop(0, fields, body, 0)

    return emb_kernel


def kernel(inputs, table):
    batch, fields = inputs.shape
    vocab, dim = table.shape
    idx_t = inputs.T.reshape(-1).astype(jnp.int32)
    emb = _make_kernel(batch, fields, vocab, dim)
    out1d = emb(idx_t, table)
    t5 = out1d.reshape(fields, dim // 8, batch // 128, 8, 128)
    return t5.transpose(2, 4, 0, 1, 3).reshape(batch, fields, dim)


# speedup vs baseline: 1.2660x; 1.2660x over previous
import functools

import jax
import jax.numpy as jnp
from jax import lax
from jax.experimental import pallas as pl
from jax.experimental.pallas import tpu as pltpu
from jax.experimental.pallas import tpu_sc as plsc


@functools.lru_cache(maxsize=None)
def _make_kernel(batch: int, fields: int, vocab: int, dim: int):
    info = plsc.get_sparse_core_info()
    num_cores, num_subcores = info.num_cores, info.num_subcores
    num_workers = num_cores * num_subcores
    lanes = 16
    bt = 128
    n_btiles = batch // bt
    ct_per_w = n_btiles // num_workers
    octets = dim // 8
    groups = bt // lanes
    n_out = batch * fields * dim

    mesh = plsc.VectorSubcoreMesh(core_axis_name="c", subcore_axis_name="s")

    @functools.partial(
        pl.kernel,
        out_type=jax.ShapeDtypeStruct((n_out,), jnp.float32),
        mesh=mesh,
        scratch_types=[
            pltpu.VMEM((fields * ct_per_w * bt,), jnp.int32),
            pltpu.VMEM((ct_per_w, bt, dim), jnp.float32),
            pltpu.VMEM((ct_per_w, octets * 8 * bt), jnp.float32),
            pltpu.SemaphoreType.DMA,
        ]
        + [pltpu.SemaphoreType.DMA for _ in range(2 * ct_per_w)],
        compiler_params=pltpu.CompilerParams(
            use_tc_tiling_on_sc=False, needs_layout_passes=False
        ),
    )
    def emb_kernel(idx_hbm, table_hbm, out_hbm, idx_v, rows_v, ostage, sem_i, *sems):
        gsems, psems = sems[:ct_per_w], sems[ct_per_w:]
        wid = lax.axis_index("s") * num_cores + lax.axis_index("c")
        cb = wid * ct_per_w

        stage = []
        for f in range(fields):
            stage.append(
                pltpu.async_copy(
                    idx_hbm.at[pl.ds(f * batch + cb * bt, ct_per_w * bt)],
                    idx_v.at[pl.ds(f * ct_per_w * bt, ct_per_w * bt)],
                    sem_i,
                )
            )
        for d in stage:
            d.wait()

        iota = lax.iota(jnp.int32, lanes)

        def body(f, carry):
            gd = []
            for cc in range(ct_per_w):
                gd.append(
                    pltpu.async_copy(
                        table_hbm.at[
                            idx_v.at[pl.ds(f * ct_per_w * bt + cc * bt, bt)]
                        ],
                        rows_v.at[cc],
                        gsems[cc],
                    )
                )
            pd = []
            for cc in range(ct_per_w):
                gd[cc].wait()

                @plsc.parallel_loop(0, groups * dim, unroll=8)
                def transpose_body(i, _cc=cc):
                    g = i // dim
                    d = i % dim
                    rowvec = iota + g * lanes
                    colvec = jnp.broadcast_to(d, (lanes,))
                    v = plsc.load_gather(rows_v.at[_cc], [rowvec, colvec])
                    flat = (d // 8) * (8 * bt) + (d % 8) * bt + g * lanes
                    ostage[_cc, pl.ds(flat, lanes)] = v

                for r in range(octets):
                    off = (
                        f * (octets * n_btiles * 8 * bt)
                        + r * (n_btiles * 8 * bt)
                        + (cb + cc) * (8 * bt)
                    )
                    pd.append(
                        pltpu.async_copy(
                            ostage.at[cc, pl.ds(r * 8 * bt, 8 * bt)],
                            out_hbm.at[pl.ds(off, 8 * bt)],
                            psems[cc],
                        )
                    )
            for d in pd:
                d.wait()
            return carry

        lax.fori_loop(0, fields, body, 0)

    return emb_kernel


def kernel(inputs, table):
    batch, fields = inputs.shape
    vocab, dim = table.shape
    idx_t = inputs.T.reshape(-1).astype(jnp.int32)
    emb = _make_kernel(batch, fields, vocab, dim)
    out1d = emb(idx_t, table)
    t5 = out1d.reshape(fields, dim // 8, batch // 128, 8, 128)
    return t5.transpose(2, 4, 0, 1, 3).reshape(batch, fields, dim)
